# Initial kernel scaffold; baseline (speedup 1.0000x reference)
import jax
import jax.numpy as jnp
from jax.experimental import pallas as pl


def kernel(observed_trajectory_target, observed_trajectory_others, neighbor_mask, W_ih, W_hh, b_ih, b_hh, W1, b1, W2, b2, Wc, bc):
    raise NotImplementedError("write your pallas kernel here")


# speedup vs baseline: 2.8273x; 2.8273x over previous
import jax
import jax.numpy as jnp
from jax.experimental import pallas as pl

H = 64
IN = 2
GX, GY = 4, 4
NS = 4.0
OBS = 15
N = 1024
G = GX * GY


def _fused_kernel(target_ref, others_ref, othersT_ref, mask_ref,
                  WihT_ref, WhhT_ref, b_ref, W1T_ref, b1_ref,
                  W2T_ref, b2_ref, WcT_ref, bc_ref, out_ref):
    f32 = jnp.float32
    WihT = WihT_ref[...]
    WhhT = WhhT_ref[...]
    b = b_ref[...]

    h = jnp.zeros((1, H), f32)
    c = jnp.zeros((1, H), f32)
    for t in range(OBS):
        x = target_ref[t:t + 1, :]
        gates = (jnp.dot(x, WihT, preferred_element_type=f32)
                 + jnp.dot(h, WhhT, preferred_element_type=f32) + b)
        gi = jax.nn.sigmoid(gates[:, 0:H])
        gf = jax.nn.sigmoid(gates[:, H:2 * H])
        gg = jnp.tanh(gates[:, 2 * H:3 * H])
        go = jax.nn.sigmoid(gates[:, 3 * H:4 * H])
        c = gf * c + gi * gg
        h = go * jnp.tanh(c)

    gates_o = jnp.dot(others_ref[...], WihT, preferred_element_type=f32) + b
    co = jax.nn.sigmoid(gates_o[:, 0:H]) * jnp.tanh(gates_o[:, 2 * H:3 * H])
    ho = jax.nn.sigmoid(gates_o[:, 3 * H:4 * H]) * jnp.tanh(co)

    cell_w = NS / GX
    cell_h = NS / GY
    px = target_ref[OBS - 1:OBS, 0:1]
    py = target_ref[OBS - 1:OBS, 1:2]
    rx = othersT_ref[0:1, :] - px
    ry = othersT_ref[1:2, :] - py
    within = (jnp.abs(rx) <= NS / 2) & (jnp.abs(ry) <= NS / 2)
    cx = (rx / cell_w).astype(jnp.int32) + GX // 2
    cy = (ry / cell_h).astype(jnp.int32) + GY // 2
    inb = (cx >= 0) & (cx < GX) & (cy >= 0) & (cy < GY)
    m = within & inb & (mask_ref[...] != 0)
    idx = jnp.where(m, cy * GX + cx, 0)

    g_iota = jax.lax.broadcasted_iota(jnp.int32, (G, N), 0)
    onehotT = ((idx == g_iota) & m).astype(f32)
    grid = jnp.dot(onehotT, ho, preferred_element_type=f32)

    acc = b1_ref[...]
    for gi_ in range(G):
        acc = acc + jnp.dot(grid[gi_:gi_ + 1, :],
                            W1T_ref[gi_ * H:(gi_ + 1) * H, :],
                            preferred_element_type=f32)
    sc = (jnp.dot(jnp.maximum(acc, 0.0), W2T_ref[...],
                  preferred_element_type=f32) + b2_ref[...])

    combined = h + sc
    out_ref[...] = (jnp.dot(combined, WcT_ref[...], preferred_element_type=f32)
                    + bc_ref[...])


def kernel(observed_trajectory_target, observed_trajectory_others, neighbor_mask,
           W_ih, W_hh, b_ih, b_hh, W1, b1, W2, b2, Wc, bc):
    others_last = observed_trajectory_others[OBS - 1]
    othersT_last = others_last.T
    mask_last = neighbor_mask[OBS - 1].reshape(1, N)
    b_comb = (b_ih + b_hh).reshape(1, 4 * H)
    out = pl.pallas_call(
        _fused_kernel,
        out_shape=jax.ShapeDtypeStruct((1, 2), jnp.float32),
    )(observed_trajectory_target, others_last, othersT_last, mask_last,
      W_ih.T, W_hh.T, b_comb, W1.T, b1.reshape(1, H),
      W2.T, b2.reshape(1, H), Wc.T, bc.reshape(1, 2))
    return out
